# baseline (speedup 1.0000x reference)
import jax
import jax.numpy as jnp
from jax.experimental import pallas as pl

B, S, H, E, TOPK = 4, 2048, 1024, 16, 2
M = B * S
TILE_M = 512


def _router_kernel(x_ref, w1_ref, b1_ref, w2t_ref, b2t_ref,
                   idx_ref, probs_ref, psum_ref, aux_ref):
    i = pl.program_id(0)
    nsteps = pl.num_programs(0)

    h = jnp.dot(x_ref[:], w1_ref[:], preferred_element_type=jnp.float32)
    h = jnp.maximum(h + b1_ref[:], 0.0)
    lt = jax.lax.dot_general(w2t_ref[:], h, (((1,), (1,)), ((), ())),
                             preferred_element_type=jnp.float32)
    lt = lt + b2t_ref[:]

    cmax = jnp.max(lt, axis=0, keepdims=True)
    ex = jnp.exp(lt - cmax)
    p = ex / jnp.sum(ex, axis=0, keepdims=True)

    iota = jax.lax.broadcasted_iota(jnp.int32, (E, TILE_M), 0)
    m1 = jnp.max(p, axis=0, keepdims=True)
    i1 = jnp.min(jnp.where(p == m1, iota, E), axis=0, keepdims=True)
    pm = jnp.where(iota == i1, -jnp.inf, p)
    m2 = jnp.max(pm, axis=0, keepdims=True)
    i2 = jnp.min(jnp.where(pm == m2, iota, E), axis=0, keepdims=True)

    denom = m1 + m2
    rows = jnp.concatenate(
        [m1 / denom, m2 / denom,
         i1.astype(jnp.float32), i2.astype(jnp.float32),
         jnp.zeros((4, TILE_M), jnp.float32)], axis=0)
    rows_t = rows.T
    probs_ref[:] = rows_t[:, 0:TOPK]
    idx_ref[:] = rows_t[:, TOPK:2 * TOPK].astype(jnp.int32)

    @pl.when(i == 0)
    def _init():
        psum_ref[:] = jnp.zeros_like(psum_ref)

    psum_ref[:, 0:1] += jnp.sum(p, axis=1, keepdims=True)

    @pl.when(i == nsteps - 1)
    def _finalize():
        rppe = psum_ref[:, 0:1] * (1.0 / M)
        aux_ref[:] = jnp.sum(rppe * jnp.log(rppe * E + 1e-9),
                             axis=0, keepdims=True)


def kernel(x, W1, b1, W2, b2):
    x2d = x.reshape(M, H)
    b1r = b1.reshape(1, H)
    w2t = W2.T
    b2t = b2.reshape(E, 1)
    grid = (M // TILE_M,)
    idx, probs, _psum, aux = pl.pallas_call(
        _router_kernel,
        grid=grid,
        in_specs=[
            pl.BlockSpec((TILE_M, H), lambda i: (i, 0)),
            pl.BlockSpec((H, H), lambda i: (0, 0)),
            pl.BlockSpec((1, H), lambda i: (0, 0)),
            pl.BlockSpec((E, H), lambda i: (0, 0)),
            pl.BlockSpec((E, 1), lambda i: (0, 0)),
        ],
        out_specs=[
            pl.BlockSpec((TILE_M, TOPK), lambda i: (i, 0)),
            pl.BlockSpec((TILE_M, TOPK), lambda i: (i, 0)),
            pl.BlockSpec((E, 1), lambda i: (0, 0)),
            pl.BlockSpec((1, 1), lambda i: (0, 0)),
        ],
        out_shape=[
            jax.ShapeDtypeStruct((M, TOPK), jnp.int32),
            jax.ShapeDtypeStruct((M, TOPK), jnp.float32),
            jax.ShapeDtypeStruct((E, 1), jnp.float32),
            jax.ShapeDtypeStruct((1, 1), jnp.float32),
        ],
    )(x2d, W1, b1r, w2t, b2t)
    return (idx.reshape(B, S, TOPK), probs.reshape(B, S, TOPK), aux[0, 0])


# speedup vs baseline: 1.0585x; 1.0585x over previous
import jax
import jax.numpy as jnp
from jax.experimental import pallas as pl

B, S, H, E, TOPK = 4, 2048, 1024, 16, 2
M = B * S
TILE_M = 4096


def _router_kernel(x_ref, w1_ref, b1_ref, w2t_ref, b2t_ref,
                   idx_ref, probs_ref, psum_ref, aux_ref):
    i = pl.program_id(0)
    nsteps = pl.num_programs(0)

    h = jnp.dot(x_ref[:], w1_ref[:], preferred_element_type=jnp.float32)
    h = jnp.maximum(h + b1_ref[:], 0.0)
    lt = jax.lax.dot_general(w2t_ref[:], h, (((1,), (1,)), ((), ())),
                             preferred_element_type=jnp.float32)
    lt = lt + b2t_ref[:]

    cmax = jnp.max(lt, axis=0, keepdims=True)
    ex = jnp.exp(lt - cmax)
    p = ex / jnp.sum(ex, axis=0, keepdims=True)

    iota = jax.lax.broadcasted_iota(jnp.int32, (E, TILE_M), 0)
    m1 = jnp.max(p, axis=0, keepdims=True)
    i1 = jnp.min(jnp.where(p == m1, iota, E), axis=0, keepdims=True)
    pm = jnp.where(iota == i1, -jnp.inf, p)
    m2 = jnp.max(pm, axis=0, keepdims=True)
    i2 = jnp.min(jnp.where(pm == m2, iota, E), axis=0, keepdims=True)

    denom = m1 + m2
    rows = jnp.concatenate(
        [m1 / denom, m2 / denom,
         i1.astype(jnp.float32), i2.astype(jnp.float32),
         jnp.zeros((4, TILE_M), jnp.float32)], axis=0)
    rows_t = rows.T
    probs_ref[:] = rows_t[:, 0:TOPK]
    idx_ref[:] = rows_t[:, TOPK:2 * TOPK].astype(jnp.int32)

    @pl.when(i == 0)
    def _init():
        psum_ref[:] = jnp.zeros_like(psum_ref)

    psum_ref[:, 0:1] += jnp.sum(p, axis=1, keepdims=True)

    @pl.when(i == nsteps - 1)
    def _finalize():
        rppe = psum_ref[:, 0:1] * (1.0 / M)
        aux_ref[:] = jnp.sum(rppe * jnp.log(rppe * E + 1e-9),
                             axis=0, keepdims=True)


def kernel(x, W1, b1, W2, b2):
    x2d = x.reshape(M, H)
    b1r = b1.reshape(1, H)
    w2t = W2.T
    b2t = b2.reshape(E, 1)
    grid = (M // TILE_M,)
    idx, probs, _psum, aux = pl.pallas_call(
        _router_kernel,
        grid=grid,
        in_specs=[
            pl.BlockSpec((TILE_M, H), lambda i: (i, 0)),
            pl.BlockSpec((H, H), lambda i: (0, 0)),
            pl.BlockSpec((1, H), lambda i: (0, 0)),
            pl.BlockSpec((E, H), lambda i: (0, 0)),
            pl.BlockSpec((E, 1), lambda i: (0, 0)),
        ],
        out_specs=[
            pl.BlockSpec((TILE_M, TOPK), lambda i: (i, 0)),
            pl.BlockSpec((TILE_M, TOPK), lambda i: (i, 0)),
            pl.BlockSpec((E, 1), lambda i: (0, 0)),
            pl.BlockSpec((1, 1), lambda i: (0, 0)),
        ],
        out_shape=[
            jax.ShapeDtypeStruct((M, TOPK), jnp.int32),
            jax.ShapeDtypeStruct((M, TOPK), jnp.float32),
            jax.ShapeDtypeStruct((E, 1), jnp.float32),
            jax.ShapeDtypeStruct((1, 1), jnp.float32),
        ],
    )(x2d, W1, b1r, w2t, b2t)
    return (idx.reshape(B, S, TOPK), probs.reshape(B, S, TOPK), aux[0, 0])


# speedup vs baseline: 1.1199x; 1.0580x over previous
import jax
import jax.numpy as jnp
from jax.experimental import pallas as pl

B, S, H, E, TOPK = 4, 2048, 1024, 16, 2
M = B * S
TILE_M = 2048


def _router_kernel(x_ref, w1_ref, b1_ref, w2t_ref, b2t_ref,
                   idx_ref, probs_ref, psum_ref, aux_ref):
    i = pl.program_id(0)
    nsteps = pl.num_programs(0)

    h = jnp.dot(x_ref[:], w1_ref[:], preferred_element_type=jnp.float32)
    h = jnp.maximum(h + b1_ref[:], 0.0)
    lt = jax.lax.dot_general(w2t_ref[:], h, (((1,), (1,)), ((), ())),
                             preferred_element_type=jnp.float32)
    lt = lt + b2t_ref[:]

    cmax = jnp.max(lt, axis=0, keepdims=True)
    ex = jnp.exp(lt - cmax)
    p = ex / jnp.sum(ex, axis=0, keepdims=True)

    iota = jax.lax.broadcasted_iota(jnp.int32, (E, TILE_M), 0)
    m1 = jnp.max(p, axis=0, keepdims=True)
    i1 = jnp.min(jnp.where(p == m1, iota, E), axis=0, keepdims=True)
    pm = jnp.where(iota == i1, -jnp.inf, p)
    m2 = jnp.max(pm, axis=0, keepdims=True)
    i2 = jnp.min(jnp.where(pm == m2, iota, E), axis=0, keepdims=True)

    denom = m1 + m2
    rows = jnp.concatenate(
        [m1 / denom, m2 / denom,
         i1.astype(jnp.float32), i2.astype(jnp.float32),
         jnp.zeros((4, TILE_M), jnp.float32)], axis=0)
    rows_t = rows.T
    probs_ref[:] = rows_t[:, 0:TOPK]
    idx_ref[:] = rows_t[:, TOPK:2 * TOPK].astype(jnp.int32)

    @pl.when(i == 0)
    def _init():
        psum_ref[:] = jnp.zeros_like(psum_ref)

    psum_ref[:, 0:1] += jnp.sum(p, axis=1, keepdims=True)

    @pl.when(i == nsteps - 1)
    def _finalize():
        rppe = psum_ref[:, 0:1] * (1.0 / M)
        aux_ref[:] = jnp.sum(rppe * jnp.log(rppe * E + 1e-9),
                             axis=0, keepdims=True)


def kernel(x, W1, b1, W2, b2):
    x2d = x.reshape(M, H)
    b1r = b1.reshape(1, H)
    w2t = W2.T
    b2t = b2.reshape(E, 1)
    grid = (M // TILE_M,)
    idx, probs, _psum, aux = pl.pallas_call(
        _router_kernel,
        grid=grid,
        in_specs=[
            pl.BlockSpec((TILE_M, H), lambda i: (i, 0)),
            pl.BlockSpec((H, H), lambda i: (0, 0)),
            pl.BlockSpec((1, H), lambda i: (0, 0)),
            pl.BlockSpec((E, H), lambda i: (0, 0)),
            pl.BlockSpec((E, 1), lambda i: (0, 0)),
        ],
        out_specs=[
            pl.BlockSpec((TILE_M, TOPK), lambda i: (i, 0)),
            pl.BlockSpec((TILE_M, TOPK), lambda i: (i, 0)),
            pl.BlockSpec((E, 1), lambda i: (0, 0)),
            pl.BlockSpec((1, 1), lambda i: (0, 0)),
        ],
        out_shape=[
            jax.ShapeDtypeStruct((M, TOPK), jnp.int32),
            jax.ShapeDtypeStruct((M, TOPK), jnp.float32),
            jax.ShapeDtypeStruct((E, 1), jnp.float32),
            jax.ShapeDtypeStruct((1, 1), jnp.float32),
        ],
    )(x2d, W1, b1r, w2t, b2t)
    return (idx.reshape(B, S, TOPK), probs.reshape(B, S, TOPK), aux[0, 0])
